# baseline (speedup 1.0000x reference)
import jax
import jax.numpy as jnp
from jax import lax
from jax.experimental import pallas as pl
from jax.experimental.pallas import tpu as pltpu
from jax.experimental.pallas import tpu_sc as plsc

B = 16384
S = 200
D = 16
C = 8
VOCAB = 1000000

NC = 2
NS = 16
NW = NC * NS
CH = 128
NG = B // CH
GPW = NG // NW
NBATCH = 8
BW = S // NBATCH

VB = 65536
NOUT = 16 * (VB // 16)


def _proj_pack_body(w_ref, t_ref, b_ref, o_ref):
    blk = t_ref[...]
    pj = (jnp.dot(w_ref[...], blk, preferred_element_type=jnp.float32)
          + b_ref[...]) * jnp.float32(1.0 / S)
    s = pj.reshape(C, VB // 2048, 16, 128)
    s2 = jnp.transpose(s, (1, 2, 0, 3))
    s3 = s2.reshape(VB // 2048, 128, 128)
    s4 = jnp.transpose(s3, (0, 2, 1))
    o_ref[...] = s4.reshape(VB // 16, 128)


def _proj_pack(wT, tt, b2):
    return pl.pallas_call(
        _proj_pack_body,
        grid=(pl.cdiv(VOCAB, VB),),
        in_specs=[
            pl.BlockSpec((C, D), lambda i: (0, 0)),
            pl.BlockSpec((D, VB), lambda i: (0, i)),
            pl.BlockSpec((C, 1), lambda i: (0, 0)),
        ],
        out_specs=pl.BlockSpec((VB // 16, 128), lambda i: (i, 0)),
        out_shape=jax.ShapeDtypeStruct((NOUT, 128), jnp.float32),
    )(wT, tt, b2)


XROWS = 40


def _x_relayout_body(xt_ref, o_ref):
    v = xt_ref[...].reshape(XROWS * 128, 128)
    o_ref[...] = (v & ~2047) | ((v & 127) << 4) | ((v >> 7) & 15)


def _x_relayout(xt):
    return pl.pallas_call(
        _x_relayout_body,
        grid=(S // XROWS,),
        in_specs=[pl.BlockSpec((XROWS, B), lambda i: (i, 0))],
        out_specs=pl.BlockSpec((XROWS * 128, 128), lambda i: (i, 0)),
        out_shape=jax.ShapeDtypeStruct((S * B // 128, 128), jnp.int32),
    )(xt)


def _sc_pool_body(x3_hbm, proj_hbm, zeros_hbm, out_hbm, idx_v, acc0, acc1, accT,
                  sem0, sem1):
    wid = lax.axis_index("s") * NC + lax.axis_index("c")
    sems = (sem0, sem1)
    accs = (acc0, acc1)
    lane = lax.iota(jnp.int32, 16)

    def group_body(ci, _):
        g = wid * GPW + ci
        pltpu.sync_copy(x3_hbm.at[:, g, :], idx_v)
        pltpu.sync_copy(zeros_hbm, acc0)
        pltpu.sync_copy(zeros_hbm, acc1)

        def fire_batch(bb):
            return [
                pltpu.async_copy(
                    proj_hbm.at[idx_v.at[bb * BW + j]],
                    accs[bb % 2],
                    sems[bb % 2],
                    add=True,
                )
                for j in range(BW)
            ]

        pend = [fire_batch(0), fire_batch(1)]
        for bb in range(2, NBATCH):
            for cp in pend[bb % 2]:
                cp.wait()
            pend[bb % 2] = fire_batch(bb)
        for cp in pend[0]:
            cp.wait()
        for cp in pend[1]:
            cp.wait()

        for k in range(C):
            kvec = jnp.full((16,), k, jnp.int32)
            for j in range(CH // 16):
                idxs = [lane + 16 * j, kvec]
                accT[k, pl.ds(16 * j, 16)] = (
                    plsc.load_gather(acc0, idxs) + plsc.load_gather(acc1, idxs)
                )
        pltpu.sync_copy(accT, out_hbm.at[:, pl.ds(g * CH, CH)])
        return 0

    lax.fori_loop(0, GPW, group_body, 0)


@jax.jit
def _sc_pool(x3, proj, zeros):
    mesh = plsc.VectorSubcoreMesh(core_axis_name="c", subcore_axis_name="s")
    return pl.kernel(
        _sc_pool_body,
        out_type=jax.ShapeDtypeStruct((C, B), jnp.float32),
        mesh=mesh,
        scratch_types=[
            pltpu.VMEM((S, CH), jnp.int32),
            pltpu.VMEM((CH, C), jnp.float32),
            pltpu.VMEM((CH, C), jnp.float32),
            pltpu.VMEM((C, CH), jnp.float32),
            pltpu.SemaphoreType.DMA,
            pltpu.SemaphoreType.DMA,
        ],
        compiler_params=pltpu.CompilerParams(
            use_tc_tiling_on_sc=False, needs_layout_passes=False
        ),
    )(x3, proj, zeros)


def kernel(x, table, W, b):
    x3 = _x_relayout(x.T).reshape(S, B // CH, CH)
    proj = _proj_pack(W.T, table.T, b.reshape(C, 1)).reshape(NOUT * 16, C)
    zeros = jnp.zeros((CH, C), jnp.float32)
    return _sc_pool(x3, proj, zeros).T


# speedup vs baseline: 1.0059x; 1.0059x over previous
import jax
import jax.numpy as jnp
from jax import lax
from jax.experimental import pallas as pl
from jax.experimental.pallas import tpu as pltpu
from jax.experimental.pallas import tpu_sc as plsc

B = 16384
S = 200
D = 16
C = 8
VOCAB = 1000000

NC = 2
NS = 16
NW = NC * NS
CH = 128
NG = B // CH
GPW = NG // NW
NBATCH = 4
BW = S // NBATCH

VB = 65536
NOUT = 16 * (VB // 16)


def _proj_pack_body(w_ref, t_ref, b_ref, o_ref):
    blk = t_ref[...]
    pj = (jnp.dot(w_ref[...], blk, preferred_element_type=jnp.float32)
          + b_ref[...]) * jnp.float32(1.0 / S)
    s = pj.reshape(C, VB // 2048, 16, 128)
    s2 = jnp.transpose(s, (1, 2, 0, 3))
    s3 = s2.reshape(VB // 2048, 128, 128)
    s4 = jnp.transpose(s3, (0, 2, 1))
    o_ref[...] = s4.reshape(VB // 16, 128)


def _proj_pack(wT, tt, b2):
    return pl.pallas_call(
        _proj_pack_body,
        grid=(pl.cdiv(VOCAB, VB),),
        in_specs=[
            pl.BlockSpec((C, D), lambda i: (0, 0)),
            pl.BlockSpec((D, VB), lambda i: (0, i)),
            pl.BlockSpec((C, 1), lambda i: (0, 0)),
        ],
        out_specs=pl.BlockSpec((VB // 16, 128), lambda i: (i, 0)),
        out_shape=jax.ShapeDtypeStruct((NOUT, 128), jnp.float32),
    )(wT, tt, b2)


XROWS = 40


def _x_relayout_body(xt_ref, o_ref):
    v = xt_ref[...].reshape(XROWS * 128, 128)
    o_ref[...] = (v & ~2047) | ((v & 127) << 4) | ((v >> 7) & 15)


def _x_relayout(xt):
    return pl.pallas_call(
        _x_relayout_body,
        grid=(S // XROWS,),
        in_specs=[pl.BlockSpec((XROWS, B), lambda i: (i, 0))],
        out_specs=pl.BlockSpec((XROWS * 128, 128), lambda i: (i, 0)),
        out_shape=jax.ShapeDtypeStruct((S * B // 128, 128), jnp.int32),
    )(xt)


def _sc_pool_body(x3_hbm, proj_hbm, zeros_hbm, out_hbm, idx_v, acc, accT,
                  sem0, sem1):
    wid = lax.axis_index("s") * NC + lax.axis_index("c")
    sems = (sem0, sem1)
    lane = lax.iota(jnp.int32, 16)

    def group_body(ci, _):
        g = wid * GPW + ci
        pltpu.sync_copy(x3_hbm.at[:, g, :], idx_v)
        pltpu.sync_copy(zeros_hbm, acc)

        def fire_batch(bb):
            return [
                pltpu.async_copy(
                    proj_hbm.at[idx_v.at[bb * BW + j]],
                    acc,
                    sems[bb % 2],
                    add=True,
                )
                for j in range(BW)
            ]

        pend = [fire_batch(0), fire_batch(1)]
        for bb in range(2, NBATCH):
            for cp in pend[bb % 2]:
                cp.wait()
            pend[bb % 2] = fire_batch(bb)
        for cp in pend[0]:
            cp.wait()
        for cp in pend[1]:
            cp.wait()

        for k in range(C):
            kvec = jnp.full((16,), k, jnp.int32)
            for j in range(CH // 16):
                accT[k, pl.ds(16 * j, 16)] = plsc.load_gather(
                    acc, [lane + 16 * j, kvec]
                )
        pltpu.sync_copy(accT, out_hbm.at[:, pl.ds(g * CH, CH)])
        return 0

    lax.fori_loop(0, GPW, group_body, 0)


@jax.jit
def _sc_pool(x3, proj, zeros):
    mesh = plsc.VectorSubcoreMesh(core_axis_name="c", subcore_axis_name="s")
    return pl.kernel(
        _sc_pool_body,
        out_type=jax.ShapeDtypeStruct((C, B), jnp.float32),
        mesh=mesh,
        scratch_types=[
            pltpu.VMEM((S, CH), jnp.int32),
            pltpu.VMEM((CH, C), jnp.float32),
            pltpu.VMEM((C, CH), jnp.float32),
            pltpu.SemaphoreType.DMA,
            pltpu.SemaphoreType.DMA,
        ],
        compiler_params=pltpu.CompilerParams(
            use_tc_tiling_on_sc=False, needs_layout_passes=False
        ),
    )(x3, proj, zeros)


def kernel(x, table, W, b):
    x3 = _x_relayout(x.T).reshape(S, B // CH, CH)
    proj = _proj_pack(W.T, table.T, b.reshape(C, 1)).reshape(NOUT * 16, C)
    zeros = jnp.zeros((CH, C), jnp.float32)
    return _sc_pool(x3, proj, zeros).T


# speedup vs baseline: 1.0215x; 1.0156x over previous
import jax
import jax.numpy as jnp
from jax import lax
from jax.experimental import pallas as pl
from jax.experimental.pallas import tpu as pltpu
from jax.experimental.pallas import tpu_sc as plsc

B = 16384
S = 200
D = 16
C = 8
VOCAB = 1000000

NC = 2
NS = 16
NW = NC * NS
CH = 128
NG = B // CH
GPW = NG // NW
NBATCH = 8
BW = S // NBATCH

VB = 65536
NOUT = 16 * (VB // 16)


def _proj_pack_body(w_ref, t_ref, b_ref, o_ref):
    blk = t_ref[...]
    pj = (jnp.dot(w_ref[...], blk, preferred_element_type=jnp.float32)
          + b_ref[...]) * jnp.float32(1.0 / S)
    s = pj.reshape(C, VB // 2048, 16, 128)
    s2 = jnp.transpose(s, (1, 2, 0, 3))
    s3 = s2.reshape(VB // 2048, 128, 128)
    s4 = jnp.transpose(s3, (0, 2, 1))
    o_ref[...] = s4.reshape(VB // 16, 128)


def _proj_pack(wT, tt, b2):
    return pl.pallas_call(
        _proj_pack_body,
        grid=(pl.cdiv(VOCAB, VB),),
        in_specs=[
            pl.BlockSpec((C, D), lambda i: (0, 0)),
            pl.BlockSpec((D, VB), lambda i: (0, i)),
            pl.BlockSpec((C, 1), lambda i: (0, 0)),
        ],
        out_specs=pl.BlockSpec((VB // 16, 128), lambda i: (i, 0)),
        out_shape=jax.ShapeDtypeStruct((NOUT, 128), jnp.float32),
    )(wT, tt, b2)


XROWS = 40


def _x_relayout_body(xt_ref, o_ref):
    v = xt_ref[...].reshape(XROWS * 128, 128)
    o_ref[...] = (v & ~2047) | ((v & 127) << 4) | ((v >> 7) & 15)


def _x_relayout(xt):
    return pl.pallas_call(
        _x_relayout_body,
        grid=(S // XROWS,),
        in_specs=[pl.BlockSpec((XROWS, B), lambda i: (i, 0))],
        out_specs=pl.BlockSpec((XROWS * 128, 128), lambda i: (i, 0)),
        out_shape=jax.ShapeDtypeStruct((S * B // 128, 128), jnp.int32),
    )(xt)


def _sc_pool_body(x3_hbm, proj_hbm, zeros_hbm, out_hbm, idx_v, acc, accT,
                  sem0, sem1):
    wid = lax.axis_index("s") * NC + lax.axis_index("c")
    sems = (sem0, sem1)
    lane = lax.iota(jnp.int32, 16)

    def group_body(ci, _):
        g = wid * GPW + ci
        pltpu.sync_copy(x3_hbm.at[:, g, :], idx_v)
        pltpu.sync_copy(zeros_hbm, acc)

        def fire_batch(bb):
            return [
                pltpu.async_copy(
                    proj_hbm.at[idx_v.at[bb * BW + j]],
                    acc,
                    sems[bb % 2],
                    add=True,
                )
                for j in range(BW)
            ]

        pend = [fire_batch(0), fire_batch(1)]
        for bb in range(2, NBATCH):
            for cp in pend[bb % 2]:
                cp.wait()
            pend[bb % 2] = fire_batch(bb)
        for cp in pend[0]:
            cp.wait()
        for cp in pend[1]:
            cp.wait()

        for k in range(C):
            kvec = jnp.full((16,), k, jnp.int32)
            for j in range(CH // 16):
                accT[k, pl.ds(16 * j, 16)] = plsc.load_gather(
                    acc, [lane + 16 * j, kvec]
                )
        pltpu.sync_copy(accT, out_hbm.at[:, pl.ds(g * CH, CH)])
        return 0

    lax.fori_loop(0, GPW, group_body, 0)


@jax.jit
def _sc_pool(x3, proj, zeros):
    mesh = plsc.VectorSubcoreMesh(core_axis_name="c", subcore_axis_name="s")
    return pl.kernel(
        _sc_pool_body,
        out_type=jax.ShapeDtypeStruct((C, B), jnp.float32),
        mesh=mesh,
        scratch_types=[
            pltpu.VMEM((S, CH), jnp.int32),
            pltpu.VMEM((CH, C), jnp.float32),
            pltpu.VMEM((C, CH), jnp.float32),
            pltpu.SemaphoreType.DMA,
            pltpu.SemaphoreType.DMA,
        ],
        compiler_params=pltpu.CompilerParams(
            use_tc_tiling_on_sc=False, needs_layout_passes=False
        ),
    )(x3, proj, zeros)


def kernel(x, table, W, b):
    x3 = _x_relayout(x.T).reshape(S, B // CH, CH)
    proj = _proj_pack(W.T, table.T, b.reshape(C, 1)).reshape(NOUT * 16, C)
    zeros = jnp.zeros((CH, C), jnp.float32)
    return _sc_pool(x3, proj, zeros).T
